# baseline (speedup 1.0000x reference)
import functools

import jax
import jax.numpy as jnp
from jax import lax
from jax.experimental import pallas as pl
from jax.experimental.pallas import tpu as pltpu
from jax.experimental.pallas import tpu_sc as plsc

CTX = 200
EMBED = 128
K = CTX * EMBED
HID = 128
NTRANS = 100000

TPB = 40
KBN = CTX // TPB
BN = 13312
NBN = -(-NTRANS // BN)
LGR = 16


def _sc_gather(emb, idx):
    info = plsc.get_sparse_core_info()
    bpw = 16
    nfull = CTX // bpw
    rem = CTX - nfull * bpw
    d = emb.shape[1]
    mesh = plsc.VectorSubcoreMesh(core_axis_name="c", subcore_axis_name="s",
                                  num_cores=1)

    @functools.partial(
        pl.kernel,
        mesh=mesh,
        out_type=jax.ShapeDtypeStruct((CTX, d), jnp.float32),
        scratch_types=[
            pltpu.VMEM((bpw,), jnp.int32),
            pltpu.VMEM((bpw, d), jnp.float32),
            pltpu.VMEM((rem,), jnp.int32),
            pltpu.VMEM((rem, d), jnp.float32),
            pltpu.SemaphoreType.DMA,
        ],
        compiler_params=pltpu.CompilerParams(use_tc_tiling_on_sc=True),
    )
    def gather_k(emb_hbm, idx_hbm, out_hbm, idx_v, rows_v, idx_r, rows_r,
                 sem):
        wid = lax.axis_index("s")

        @pl.when(wid < nfull)
        def _():
            base = wid * bpw
            pltpu.sync_copy(idx_hbm.at[pl.ds(base, bpw)], idx_v)
            pltpu.async_copy(emb_hbm.at[idx_v], rows_v, sem).wait()
            pltpu.sync_copy(rows_v, out_hbm.at[pl.ds(base, bpw)])

        @pl.when(wid == nfull)
        def _():
            base = nfull * bpw
            pltpu.sync_copy(idx_hbm.at[pl.ds(base, rem)], idx_r)
            pltpu.async_copy(emb_hbm.at[idx_r], rows_r, sem).wait()
            pltpu.sync_copy(rows_r, out_hbm.at[pl.ds(base, rem)])

    return gather_k(emb, idx)


def _mlp_body(e_ref, w1_ref, b1_ref, w2_ref, b2_ref, out_ref,
              h_ref, m_ref, s_ref, lg_ref):
    i = pl.program_id(0)

    @pl.when(i == 0)
    def _init():
        h_ref[...] = jnp.zeros_like(h_ref)

    @pl.when(i < KBN)
    def _phase_a():
        acc0 = h_ref[...]
        acc1 = jnp.zeros((1, HID), jnp.float32)
        for t in range(0, TPB, 2):
            acc0 += jnp.dot(e_ref[t:t + 1, :], w1_ref[t],
                            preferred_element_type=jnp.float32)
            acc1 += jnp.dot(e_ref[t + 1:t + 2, :], w1_ref[t + 1],
                            preferred_element_type=jnp.float32)
        h_ref[...] = acc0 + acc1

    @pl.when(i == KBN - 1)
    def _finish_h():
        h_ref[...] = jnp.maximum(h_ref[...] + b1_ref[...].reshape(1, HID),
                                 0.0)

    @pl.when((i >= KBN) & (i < KBN + NBN))
    def _phase_b():
        j = i - KBN
        o = (lax.dot_general(h_ref[...], w2_ref[...],
                             (((1,), (1,)), ((), ())),
                             preferred_element_type=jnp.float32)
             + b2_ref[...].reshape(1, BN))
        col = j * BN + lax.broadcasted_iota(jnp.int32, (1, BN), 1)
        o = jnp.where(col < NTRANS, o, -jnp.inf)
        row = lax.broadcasted_iota(jnp.int32, (LGR, BN), 0)
        lg_ref[...] = jnp.where(row == j, o, lg_ref[...])
        bm = jnp.max(o)
        bs = jnp.sum(jnp.where(col < NTRANS, jnp.exp(o - bm), 0.0))
        bm_v = jnp.full((1, HID), bm, jnp.float32)
        bs_v = jnp.full((1, HID), bs, jnp.float32)

        @pl.when(j == 0)
        def _first():
            m_ref[...] = bm_v
            s_ref[...] = bs_v

        @pl.when(j > 0)
        def _combine():
            m_old = m_ref[...]
            m_new = jnp.maximum(m_old, bm_v)
            s_ref[...] = (s_ref[...] * jnp.exp(m_old - m_new)
                          + bs_v * jnp.exp(bm_v - m_new))
            m_ref[...] = m_new

        @pl.when(i == KBN + NBN - 1)
        def _final():
            m_ref[...] = m_ref[...] + jnp.log(s_ref[...])

    @pl.when(i >= KBN + NBN)
    def _phase_c():
        c = i - KBN - NBN
        row = lax.broadcasted_iota(jnp.int32, (LGR, BN), 0)
        picked = jnp.sum(jnp.where(row == c, lg_ref[...], 0.0), axis=0,
                         keepdims=True)
        out_ref[...] = picked - m_ref[0, 0]


def _mlp_logprobs(rows, w1_3d, b1, w2t, b2):
    return pl.pallas_call(
        _mlp_body,
        grid=(KBN + 2 * NBN,),
        in_specs=[
            pl.BlockSpec((TPB, EMBED), lambda i: (jnp.minimum(i, KBN - 1), 0)),
            pl.BlockSpec((TPB, EMBED, HID),
                         lambda i: (jnp.minimum(i, KBN - 1), 0, 0)),
            pl.BlockSpec((HID,), lambda i: (0,)),
            pl.BlockSpec((BN, HID),
                         lambda i: (jnp.clip(i - KBN, 0, NBN - 1), 0)),
            pl.BlockSpec((BN,),
                         lambda i: (jnp.clip(i - KBN, 0, NBN - 1),)),
        ],
        out_specs=pl.BlockSpec((1, BN),
                               lambda i: (0, jnp.maximum(i - KBN - NBN, 0))),
        out_shape=jax.ShapeDtypeStruct((1, NTRANS), jnp.float32),
        scratch_shapes=[
            pltpu.VMEM((1, HID), jnp.float32),
            pltpu.VMEM((1, HID), jnp.float32),
            pltpu.VMEM((1, HID), jnp.float32),
            pltpu.VMEM((LGR, BN), jnp.float32),
        ],
    )(rows, w1_3d, b1, w2t, b2)


def kernel(x, emb, W1, b1, W2, b2):
    rows = _sc_gather(emb, x)
    return _mlp_logprobs(rows, W1.reshape(CTX, EMBED, HID), b1, W2.T, b2)


# speedup vs baseline: 1.0683x; 1.0683x over previous
import functools

import jax
import jax.numpy as jnp
from jax import lax
from jax.experimental import pallas as pl
from jax.experimental.pallas import tpu as pltpu
from jax.experimental.pallas import tpu_sc as plsc

CTX = 200
EMBED = 128
K = CTX * EMBED
HID = 128
NTRANS = 100000

TPB = 40
KBN = CTX // TPB
BN = 13312
NBN = -(-NTRANS // BN)
LGR = 16


def _sc_gather(emb, idx):
    info = plsc.get_sparse_core_info()
    bpw = 16
    nfull = CTX // bpw
    rem = CTX - nfull * bpw
    d = emb.shape[1]
    mesh = plsc.VectorSubcoreMesh(core_axis_name="c", subcore_axis_name="s",
                                  num_cores=1)

    @functools.partial(
        pl.kernel,
        mesh=mesh,
        out_type=jax.ShapeDtypeStruct((CTX, d), jnp.float32),
        scratch_types=[
            pltpu.VMEM((bpw,), jnp.int32),
            pltpu.VMEM((bpw, d), jnp.float32),
            pltpu.VMEM((rem,), jnp.int32),
            pltpu.VMEM((rem, d), jnp.float32),
            pltpu.SemaphoreType.DMA,
        ],
        compiler_params=pltpu.CompilerParams(use_tc_tiling_on_sc=True),
    )
    def gather_k(emb_hbm, idx_hbm, out_hbm, idx_v, rows_v, idx_r, rows_r,
                 sem):
        wid = lax.axis_index("s")

        @pl.when(wid < nfull)
        def _():
            base = wid * bpw
            pltpu.sync_copy(idx_hbm.at[pl.ds(base, bpw)], idx_v)
            pltpu.async_copy(emb_hbm.at[idx_v], rows_v, sem).wait()
            pltpu.sync_copy(rows_v, out_hbm.at[pl.ds(base, bpw)])

        @pl.when(wid == nfull)
        def _():
            base = nfull * bpw
            pltpu.sync_copy(idx_hbm.at[pl.ds(base, rem)], idx_r)
            pltpu.async_copy(emb_hbm.at[idx_r], rows_r, sem).wait()
            pltpu.sync_copy(rows_r, out_hbm.at[pl.ds(base, rem)])

    return gather_k(emb, idx)


def _mlp_body(e_ref, w1_ref, b1_ref, w2_ref, b2_ref, out_ref,
              h_ref, m_ref, s_ref, lg_ref):
    i = pl.program_id(0)

    @pl.when(i == 0)
    def _init():
        h_ref[...] = jnp.zeros_like(h_ref)

    @pl.when(i < KBN)
    def _phase_a():
        acc0 = h_ref[...]
        acc1 = jnp.zeros((1, HID), jnp.float32)
        for t in range(0, TPB, 2):
            acc0 += jnp.dot(e_ref[t:t + 1, :], w1_ref[t],
                            preferred_element_type=jnp.float32)
            acc1 += jnp.dot(e_ref[t + 1:t + 2, :], w1_ref[t + 1],
                            preferred_element_type=jnp.float32)
        h_ref[...] = acc0 + acc1

    @pl.when(i == KBN - 1)
    def _finish_h():
        h_ref[...] = jnp.maximum(h_ref[...] + b1_ref[...].reshape(1, HID),
                                 0.0)

    @pl.when((i >= KBN) & (i < KBN + NBN))
    def _phase_b():
        j = i - KBN
        o = (lax.dot_general(h_ref[...], w2_ref[...],
                             (((1,), (1,)), ((), ())),
                             preferred_element_type=jnp.float32)
             + b2_ref[...].reshape(1, BN))
        tail = NBN * BN - NTRANS
        o = jnp.where(
            (j < NBN - 1)
            | (lax.broadcasted_iota(jnp.int32, (1, BN), 1) < BN - tail),
            o, -jnp.inf)
        lg_ref[pl.ds(j, 1), :] = o
        bm = jnp.max(o)
        bs = jnp.sum(jnp.exp(o - bm))
        bm_v = jnp.full((1, HID), bm, jnp.float32)
        bs_v = jnp.full((1, HID), bs, jnp.float32)

        @pl.when(j == 0)
        def _first():
            m_ref[...] = bm_v
            s_ref[...] = bs_v

        @pl.when(j > 0)
        def _combine():
            m_old = m_ref[...]
            m_new = jnp.maximum(m_old, bm_v)
            s_ref[...] = (s_ref[...] * jnp.exp(m_old - m_new)
                          + bs_v * jnp.exp(bm_v - m_new))
            m_ref[...] = m_new

        @pl.when(i == KBN + NBN - 1)
        def _final():
            m_ref[...] = m_ref[...] + jnp.log(s_ref[...])

    @pl.when(i >= KBN + NBN)
    def _phase_c():
        c = i - KBN - NBN
        out_ref[...] = lg_ref[pl.ds(c, 1), :] - m_ref[0, 0]


def _mlp_logprobs(rows, w1_3d, b1, w2t, b2):
    return pl.pallas_call(
        _mlp_body,
        grid=(KBN + 2 * NBN,),
        in_specs=[
            pl.BlockSpec((TPB, EMBED), lambda i: (jnp.minimum(i, KBN - 1), 0)),
            pl.BlockSpec((TPB, EMBED, HID),
                         lambda i: (jnp.minimum(i, KBN - 1), 0, 0)),
            pl.BlockSpec((HID,), lambda i: (0,)),
            pl.BlockSpec((BN, HID),
                         lambda i: (jnp.clip(i - KBN, 0, NBN - 1), 0)),
            pl.BlockSpec((BN,),
                         lambda i: (jnp.clip(i - KBN, 0, NBN - 1),)),
        ],
        out_specs=pl.BlockSpec((1, BN),
                               lambda i: (0, jnp.maximum(i - KBN - NBN, 0))),
        out_shape=jax.ShapeDtypeStruct((1, NTRANS), jnp.float32),
        scratch_shapes=[
            pltpu.VMEM((1, HID), jnp.float32),
            pltpu.VMEM((1, HID), jnp.float32),
            pltpu.VMEM((1, HID), jnp.float32),
            pltpu.VMEM((LGR, BN), jnp.float32),
        ],
    )(rows, w1_3d, b1, w2t, b2)


def kernel(x, emb, W1, b1, W2, b2):
    rows = _sc_gather(emb, x)
    return _mlp_logprobs(rows, W1.reshape(CTX, EMBED, HID), b1, W2.T, b2)
